# baseline (speedup 1.0000x reference)
import functools

import jax
import jax.numpy as jnp
from jax import lax
from jax.experimental import pallas as pl
from jax.experimental.pallas import tpu as pltpu
from jax.experimental.pallas import tpu_sc as plsc

N = 10000
E = 320000
G = 128
CH = 100
NC = 2
NS = 16
NW = NC * NS
EPT = E // NW
NCH = EPT // CH
NP = 10112
RPT = NP // NS
ZCH = 80
NBUF = 3

_SC_PARAMS = pltpu.CompilerParams(use_tc_tiling_on_sc=False)
_MESH = plsc.VectorSubcoreMesh(core_axis_name="c", subcore_axis_name="s")


def _fill(buf, rows, d, val):
    v16 = jnp.full((16,), val, jnp.float32)
    for r in range(rows):
        for k in range(d // 16):
            buf[r, pl.ds(k * 16, 16)] = v16


@functools.cache
def _make_agg(d, with_deg=False):
    nbuf = 2 if with_deg else NBUF

    def body(y, pck, *outs_and_scratch):
        rest = list(outs_and_scratch)
        out = rest.pop(0)
        if with_deg:
            outd = rest.pop(0)
        acc, pck_v, srow, drow = rest[:4]
        rest = rest[4:]
        if with_deg:
            acc2, ones = rest[:2]
            rest = rest[2:]
        bufs = rest[:nbuf]
        gsems = rest[nbuf:2 * nbuf]
        c = lax.axis_index("c")
        s = lax.axis_index("s")
        w = c * NS + s
        pltpu.sync_copy(pck.at[w], pck_v)
        _fill(bufs[0], ZCH, d, 0.0)
        base = s * RPT
        off = 0
        while off < RPT:
            zn = min(ZCH, RPT - off)
            pltpu.sync_copy(bufs[0].at[pl.ds(0, zn)],
                            acc.at[pl.ds(base + off, zn)])
            off += zn
        if with_deg:
            _fill(ones, ZCH, 16, 0.0)
            off = 0
            while off < RPT:
                zn = min(ZCH, RPT - off)
                pltpu.sync_copy(ones.at[pl.ds(0, zn)],
                                acc2.at[pl.ds(base + off, zn)])
                off += zn
            _fill(ones, CH, 16, 1.0)
        plsc.subcore_barrier()

        offs = [k * 16 for k in range(CH // 16)]
        if CH % 16:
            offs.append(CH - 16)

        def fire_g(jj, b):
            for k in offs:
                v = pck_v[jj, pl.ds(k, 16)]
                srow[b, pl.ds(k, 16)] = v >> 14
                drow[b, pl.ds(k, 16)] = v & 16383
            pltpu.async_copy(y.at[srow.at[b]], bufs[b], gsems[b])

        def drain(jj, b):
            pltpu.make_async_copy(y.at[srow.at[b]], bufs[b], gsems[b]).wait()
            pltpu.sync_copy(bufs[b], acc.at[drow.at[b]], add=True)
            if with_deg:
                pltpu.sync_copy(ones, acc2.at[drow.at[b]], add=True)

        for b in range(nbuf):
            fire_g(b, b)

        M = ((NCH - nbuf) // nbuf) * nbuf

        @pl.loop(0, M, step=nbuf)
        def _chunk(j):
            for b in range(nbuf):
                drain(j + b, b)
                fire_g(j + b + nbuf, b)

        for t in range(M, NCH):
            drain(t, t % nbuf)
            if t + nbuf < NCH:
                fire_g(t + nbuf, (t + nbuf) % nbuf)

        plsc.subcore_barrier()
        pltpu.sync_copy(acc.at[pl.ds(base, RPT)], out.at[c, pl.ds(base, RPT)])
        if with_deg:
            pltpu.sync_copy(acc2.at[pl.ds(base, RPT)],
                            outd.at[c, pl.ds(base, RPT)])

    out_type = [jax.ShapeDtypeStruct((NC, NP, d), jnp.float32)]
    if with_deg:
        out_type.append(jax.ShapeDtypeStruct((NC, NP, 16), jnp.float32))
    scratch = [
        pltpu.VMEM_SHARED((NP, d), jnp.float32),
        pltpu.VMEM((NCH, CH), jnp.int32),
        pltpu.VMEM((nbuf, CH), jnp.int32),
        pltpu.VMEM((nbuf, CH), jnp.int32),
    ]
    if with_deg:
        scratch += [
            pltpu.VMEM_SHARED((NP, 16), jnp.float32),
            pltpu.VMEM((CH, 16), jnp.float32),
        ]
    scratch += [pltpu.VMEM((CH, d), jnp.float32) for _ in range(nbuf)]
    scratch += [pltpu.SemaphoreType.DMA for _ in range(nbuf)]

    return pl.kernel(
        body,
        out_type=tuple(out_type) if with_deg else out_type[0],
        mesh=_MESH,
        compiler_params=_SC_PARAMS,
        scratch_types=scratch,
    )


BR = 2000
GRID = N // BR


def _rows(dd):
    return pl.BlockSpec((BR, dd), lambda i: (i, 0))


def _prows(dd):
    return pl.BlockSpec((2, BR, dd), lambda i: (0, i, 0))


def _full(*shape):
    return pl.BlockSpec(shape, lambda i: (0,) * len(shape))


def _mid1_body(p, pd, w1, b1, w2, y2, invd):
    a = p[...]
    dg = pd[...]
    inv = 1.0 / jnp.maximum(dg[0] + dg[1], 1.0)
    agg = (a[0] + a[1]) * inv[:, :1]
    h1 = jnp.maximum(
        jnp.dot(agg, w1[...], preferred_element_type=jnp.float32)
        + b1[...][None, :], 0.0)
    y2[...] = jnp.dot(h1, w2[...], preferred_element_type=jnp.float32)
    invd[...] = inv


_mid1 = pl.pallas_call(
    _mid1_body, grid=(GRID,),
    in_specs=[_prows(G), _prows(16), _full(G, G), _full(G), _full(G, 64)],
    out_specs=(_rows(64), _rows(16)),
    out_shape=(jax.ShapeDtypeStruct((N, 64), jnp.float32),
               jax.ShapeDtypeStruct((N, 16), jnp.float32)))


def _mid2_body(p, invd, b2, w3, y3):
    a = p[...]
    h2 = jnp.maximum((a[0] + a[1]) * invd[...][:, :1] + b2[...][None, :], 0.0)
    y3[...] = jnp.dot(h2, w3[...], preferred_element_type=jnp.float32)


_mid2 = pl.pallas_call(
    _mid2_body, grid=(GRID,),
    in_specs=[_prows(64), _rows(16), _full(64), _full(64, 64)],
    out_specs=_rows(64),
    out_shape=jax.ShapeDtypeStruct((N, 64), jnp.float32))


def _mid3_body(p, invd, b31, b32, eps, mu, lv, z):
    a = p[...]
    sm = (a[0] + a[1]) * invd[...][:, :1]
    mu_ = sm[:, :32] + b31[...][None, :]
    lv_ = sm[:, 32:] + b32[...][None, :]
    mu[...] = mu_
    lv[...] = lv_
    z[...] = eps[...] * jnp.exp(0.5 * lv_) + mu_


_mid3 = pl.pallas_call(
    _mid3_body, grid=(GRID,),
    in_specs=[_prows(64), _rows(16), _full(32), _full(32), _rows(32)],
    out_specs=(_rows(32), _rows(32), _rows(32)),
    out_shape=(jax.ShapeDtypeStruct((N, 32), jnp.float32),
               jax.ShapeDtypeStruct((N, 32), jnp.float32),
               jax.ShapeDtypeStruct((N, 32), jnp.float32)))


def _make_post(din, dout, act):
    def body(p, invd, w, b, o):
        a = p[...]
        agg = (a[0] + a[1]) * invd[...][:, :1]
        o[...] = act(jnp.dot(agg, w[...], preferred_element_type=jnp.float32)
                     + b[...][None, :])

    return pl.pallas_call(
        body, grid=(GRID,),
        in_specs=[_prows(din), _rows(16), _full(din, dout), _full(dout)],
        out_specs=_rows(dout),
        out_shape=jax.ShapeDtypeStruct((N, dout), jnp.float32))


_relu = lambda t: jnp.maximum(t, 0.0)
_mid4 = _make_post(32, 64, _relu)
_mid5 = _make_post(64, G, _relu)
_fin = _make_post(G, G, jax.nn.sigmoid)


def kernel(x, edge_index, eps, W1, b1, W2, b2, W31, b31, W32, b32,
           W4, b4, W5, b5, W6, b6):
    pck = ((edge_index[0] << 14) | edge_index[1]).reshape(NW, NCH, CH)
    W3 = jnp.concatenate([W31, W32], axis=1)

    p1, pdeg = _make_agg(G, True)(x, pck)
    y2, invd = _mid1(p1, pdeg, W1, b1, W2)
    p2 = _make_agg(64)(y2, pck)
    y3 = _mid2(p2, invd, b2, W3)
    p3 = _make_agg(64)(y3, pck)
    mu, log_var, z = _mid3(p3, invd, b31, b32, eps)
    p4 = _make_agg(32)(z, pck)
    h4 = _mid4(p4, invd, W4, b4)
    p5 = _make_agg(64)(h4, pck)
    h5 = _mid5(p5, invd, W5, b5)
    p6 = _make_agg(G)(h5, pck)
    recon = _fin(p6, invd, W6, b6)
    return (x, recon, mu, log_var)


# speedup vs baseline: 1.0225x; 1.0225x over previous
import functools

import jax
import jax.numpy as jnp
from jax import lax
from jax.experimental import pallas as pl
from jax.experimental.pallas import tpu as pltpu
from jax.experimental.pallas import tpu_sc as plsc

N = 10000
E = 320000
G = 128
CH = 100
NC = 2
NS = 16
NW = NC * NS
EPT = E // NW
NCH = EPT // CH
NP = 10112
RPT = NP // NS
ZCH = 80
NBUF = 3

_SC_PARAMS = pltpu.CompilerParams(use_tc_tiling_on_sc=False)
_MESH = plsc.VectorSubcoreMesh(core_axis_name="c", subcore_axis_name="s")


def _fill(buf, rows, d, val):
    v16 = jnp.full((16,), val, jnp.float32)
    for r in range(rows):
        for k in range(d // 16):
            buf[r, pl.ds(k * 16, 16)] = v16


@functools.cache
def _make_agg(d, with_deg=False):
    nbuf = 2 if with_deg else NBUF

    def body(y, pck, *outs_and_scratch):
        rest = list(outs_and_scratch)
        out = rest.pop(0)
        if with_deg:
            outd = rest.pop(0)
        acc, pck_v, srow, drow = rest[:4]
        rest = rest[4:]
        if with_deg:
            acc2, ones = rest[:2]
            rest = rest[2:]
        bufs = rest[:nbuf]
        gsems = rest[nbuf:2 * nbuf]
        c = lax.axis_index("c")
        s = lax.axis_index("s")
        w = c * NS + s
        pltpu.sync_copy(pck.at[w], pck_v)
        _fill(bufs[0], ZCH, d, 0.0)
        base = s * RPT
        off = 0
        while off < RPT:
            zn = min(ZCH, RPT - off)
            pltpu.sync_copy(bufs[0].at[pl.ds(0, zn)],
                            acc.at[pl.ds(base + off, zn)])
            off += zn
        if with_deg:
            _fill(ones, ZCH, 16, 0.0)
            off = 0
            while off < RPT:
                zn = min(ZCH, RPT - off)
                pltpu.sync_copy(ones.at[pl.ds(0, zn)],
                                acc2.at[pl.ds(base + off, zn)])
                off += zn
            _fill(ones, CH, 16, 1.0)
        plsc.subcore_barrier()

        offs = [k * 16 for k in range(CH // 16)]
        if CH % 16:
            offs.append(CH - 16)

        def fire_g(jj, b):
            for k in offs:
                v = pck_v[jj, pl.ds(k, 16)]
                srow[b, pl.ds(k, 16)] = v >> 14
                drow[b, pl.ds(k, 16)] = v & 16383
            pltpu.async_copy(y.at[srow.at[b]], bufs[b], gsems[b])

        def drain(jj, b):
            pltpu.make_async_copy(y.at[srow.at[b]], bufs[b], gsems[b]).wait()
            pltpu.sync_copy(bufs[b], acc.at[drow.at[b]], add=True)
            if with_deg:
                pltpu.sync_copy(ones, acc2.at[drow.at[b]], add=True)

        for b in range(nbuf):
            fire_g(b, b)

        M = ((NCH - nbuf) // nbuf) * nbuf

        @pl.loop(0, M, step=nbuf)
        def _chunk(j):
            for b in range(nbuf):
                drain(j + b, b)
                fire_g(j + b + nbuf, b)

        for t in range(M, NCH):
            drain(t, t % nbuf)
            if t + nbuf < NCH:
                fire_g(t + nbuf, (t + nbuf) % nbuf)

        plsc.subcore_barrier()
        pltpu.sync_copy(acc.at[pl.ds(base, RPT)], out.at[c, pl.ds(base, RPT)])
        if with_deg:
            pltpu.sync_copy(acc2.at[pl.ds(base, RPT)],
                            outd.at[c, pl.ds(base, RPT)])

    out_type = [jax.ShapeDtypeStruct((NC, NP, d), jnp.float32)]
    if with_deg:
        out_type.append(jax.ShapeDtypeStruct((NC, NP, 16), jnp.float32))
    scratch = [
        pltpu.VMEM_SHARED((NP, d), jnp.float32),
        pltpu.VMEM((NCH, CH), jnp.int32),
        pltpu.VMEM((nbuf, CH), jnp.int32),
        pltpu.VMEM((nbuf, CH), jnp.int32),
    ]
    if with_deg:
        scratch += [
            pltpu.VMEM_SHARED((NP, 16), jnp.float32),
            pltpu.VMEM((CH, 16), jnp.float32),
        ]
    scratch += [pltpu.VMEM((CH, d), jnp.float32) for _ in range(nbuf)]
    scratch += [pltpu.SemaphoreType.DMA for _ in range(nbuf)]

    return pl.kernel(
        body,
        out_type=tuple(out_type) if with_deg else out_type[0],
        mesh=_MESH,
        compiler_params=_SC_PARAMS,
        scratch_types=scratch,
    )


def _deg_body(dst, out, acc, dst_v, ones, sem0, sem1):
    sems = (sem0, sem1)
    c = lax.axis_index("c")
    s = lax.axis_index("s")
    w = c * NS + s
    pltpu.sync_copy(dst.at[w], dst_v)
    _fill(ones, ZCH, 16, 0.0)
    base = s * RPT
    off = 0
    while off < RPT:
        zn = min(ZCH, RPT - off)
        pltpu.sync_copy(ones.at[pl.ds(0, zn)],
                        acc.at[pl.ds(base + off, zn)])
        off += zn
    _fill(ones, CH, 16, 1.0)
    plsc.subcore_barrier()

    def fire(jj, b):
        pltpu.async_copy(ones, acc.at[dst_v.at[jj]], sems[b], add=True)

    def wait(jj, b):
        pltpu.make_async_copy(ones, acc.at[dst_v.at[jj]], sems[b]).wait()

    fire(0, 0)
    fire(1, 1)

    @pl.loop(0, NCH - 2, step=2)
    def _chunk(j):
        for b in range(2):
            wait(j + b, b)
            fire(j + b + 2, b)

    wait(NCH - 2, 0)
    wait(NCH - 1, 1)
    plsc.subcore_barrier()
    pltpu.sync_copy(acc.at[pl.ds(base, RPT)], out.at[c, pl.ds(base, RPT)])


_deg = pl.kernel(
    _deg_body,
    out_type=jax.ShapeDtypeStruct((NC, NP, 16), jnp.float32),
    mesh=_MESH,
    compiler_params=_SC_PARAMS,
    scratch_types=[
        pltpu.VMEM_SHARED((NP, 16), jnp.float32),
        pltpu.VMEM((NCH, CH), jnp.int32),
        pltpu.VMEM((CH, 16), jnp.float32),
        pltpu.SemaphoreType.DMA,
        pltpu.SemaphoreType.DMA,
    ],
)


BR = 2000
GRID = N // BR


def _rows(dd):
    return pl.BlockSpec((BR, dd), lambda i: (i, 0))


def _prows(dd):
    return pl.BlockSpec((2, BR, dd), lambda i: (0, i, 0))


def _full(*shape):
    return pl.BlockSpec(shape, lambda i: (0,) * len(shape))


def _mid1_body(p, pd, w1, b1, w2, y2, invd):
    a = p[...]
    dg = pd[...]
    inv = 1.0 / jnp.maximum(dg[0] + dg[1], 1.0)
    agg = (a[0] + a[1]) * inv[:, :1]
    h1 = jnp.maximum(
        jnp.dot(agg, w1[...], preferred_element_type=jnp.float32)
        + b1[...][None, :], 0.0)
    y2[...] = jnp.dot(h1, w2[...], preferred_element_type=jnp.float32)
    invd[...] = inv


_mid1 = pl.pallas_call(
    _mid1_body, grid=(GRID,),
    in_specs=[_prows(G), _prows(16), _full(G, G), _full(G), _full(G, 64)],
    out_specs=(_rows(64), _rows(16)),
    out_shape=(jax.ShapeDtypeStruct((N, 64), jnp.float32),
               jax.ShapeDtypeStruct((N, 16), jnp.float32)))


def _mid2_body(p, invd, b2, w3, y3):
    a = p[...]
    h2 = jnp.maximum((a[0] + a[1]) * invd[...][:, :1] + b2[...][None, :], 0.0)
    y3[...] = jnp.dot(h2, w3[...], preferred_element_type=jnp.float32)


_mid2 = pl.pallas_call(
    _mid2_body, grid=(GRID,),
    in_specs=[_prows(64), _rows(16), _full(64), _full(64, 64)],
    out_specs=_rows(64),
    out_shape=jax.ShapeDtypeStruct((N, 64), jnp.float32))


def _mid3_body(p, invd, b31, b32, eps, mu, lv, z):
    a = p[...]
    sm = (a[0] + a[1]) * invd[...][:, :1]
    mu_ = sm[:, :32] + b31[...][None, :]
    lv_ = sm[:, 32:] + b32[...][None, :]
    mu[...] = mu_
    lv[...] = lv_
    z[...] = eps[...] * jnp.exp(0.5 * lv_) + mu_


_mid3 = pl.pallas_call(
    _mid3_body, grid=(GRID,),
    in_specs=[_prows(64), _rows(16), _full(32), _full(32), _rows(32)],
    out_specs=(_rows(32), _rows(32), _rows(32)),
    out_shape=(jax.ShapeDtypeStruct((N, 32), jnp.float32),
               jax.ShapeDtypeStruct((N, 32), jnp.float32),
               jax.ShapeDtypeStruct((N, 32), jnp.float32)))


def _make_post(din, dout, act):
    def body(p, invd, w, b, o):
        a = p[...]
        agg = (a[0] + a[1]) * invd[...][:, :1]
        o[...] = act(jnp.dot(agg, w[...], preferred_element_type=jnp.float32)
                     + b[...][None, :])

    return pl.pallas_call(
        body, grid=(GRID,),
        in_specs=[_prows(din), _rows(16), _full(din, dout), _full(dout)],
        out_specs=_rows(dout),
        out_shape=jax.ShapeDtypeStruct((N, dout), jnp.float32))


_relu = lambda t: jnp.maximum(t, 0.0)
_mid4 = _make_post(32, 64, _relu)
_mid5 = _make_post(64, G, _relu)
_fin = _make_post(G, G, jax.nn.sigmoid)


def kernel(x, edge_index, eps, W1, b1, W2, b2, W31, b31, W32, b32,
           W4, b4, W5, b5, W6, b6):
    dst = edge_index[1].reshape(NW, NCH, CH)
    pck = ((edge_index[0] << 14) | edge_index[1]).reshape(NW, NCH, CH)
    W3 = jnp.concatenate([W31, W32], axis=1)

    pdeg = _deg(dst)
    p1 = _make_agg(G)(x, pck)
    y2, invd = _mid1(p1, pdeg, W1, b1, W2)
    p2 = _make_agg(64)(y2, pck)
    y3 = _mid2(p2, invd, b2, W3)
    p3 = _make_agg(64)(y3, pck)
    mu, log_var, z = _mid3(p3, invd, b31, b32, eps)
    p4 = _make_agg(32)(z, pck)
    h4 = _mid4(p4, invd, W4, b4)
    p5 = _make_agg(64)(h4, pck)
    h5 = _mid5(p5, invd, W5, b5)
    p6 = _make_agg(G)(h5, pck)
    recon = _fin(p6, invd, W6, b6)
    return (x, recon, mu, log_var)


# speedup vs baseline: 1.0716x; 1.0480x over previous
import functools

import jax
import jax.numpy as jnp
from jax import lax
from jax.experimental import pallas as pl
from jax.experimental.pallas import tpu as pltpu
from jax.experimental.pallas import tpu_sc as plsc

N = 10000
E = 320000
G = 128
CH = 100
NC = 2
NS = 16
NW = NC * NS
EPT = E // NW
NCH = EPT // CH
NP = 10112
RPT = NP // NS
ZCH = 80
NBUF = 3

_SC_PARAMS = pltpu.CompilerParams(use_tc_tiling_on_sc=False)
_MESH = plsc.VectorSubcoreMesh(core_axis_name="c", subcore_axis_name="s")


def _fill(buf, rows, d, val):
    v16 = jnp.full((16,), val, jnp.float32)
    for r in range(rows):
        for k in range(d // 16):
            buf[r, pl.ds(k * 16, 16)] = v16


@functools.cache
def _make_agg(d, with_deg=False):
    nbuf = 2 if with_deg else (NBUF if d > 64 else 5)

    def body(y, pck, *outs_and_scratch):
        rest = list(outs_and_scratch)
        out = rest.pop(0)
        if with_deg:
            outd = rest.pop(0)
        acc, pck_v, srow, drow = rest[:4]
        rest = rest[4:]
        if with_deg:
            acc2, ones = rest[:2]
            rest = rest[2:]
        bufs = rest[:nbuf]
        gsems = rest[nbuf:2 * nbuf]
        c = lax.axis_index("c")
        s = lax.axis_index("s")
        w = c * NS + s
        pltpu.sync_copy(pck.at[w], pck_v)
        _fill(bufs[0], ZCH, d, 0.0)
        base = s * RPT
        off = 0
        while off < RPT:
            zn = min(ZCH, RPT - off)
            pltpu.sync_copy(bufs[0].at[pl.ds(0, zn)],
                            acc.at[pl.ds(base + off, zn)])
            off += zn
        if with_deg:
            _fill(ones, ZCH, 16, 0.0)
            off = 0
            while off < RPT:
                zn = min(ZCH, RPT - off)
                pltpu.sync_copy(ones.at[pl.ds(0, zn)],
                                acc2.at[pl.ds(base + off, zn)])
                off += zn
            _fill(ones, CH, 16, 1.0)
        plsc.subcore_barrier()

        offs = [k * 16 for k in range(CH // 16)]
        if CH % 16:
            offs.append(CH - 16)

        def fire_g(jj, b):
            for k in offs:
                v = pck_v[jj, pl.ds(k, 16)]
                srow[b, pl.ds(k, 16)] = v >> 14
                drow[b, pl.ds(k, 16)] = v & 16383
            pltpu.async_copy(y.at[srow.at[b]], bufs[b], gsems[b])

        def drain(jj, b):
            pltpu.make_async_copy(y.at[srow.at[b]], bufs[b], gsems[b]).wait()
            pltpu.sync_copy(bufs[b], acc.at[drow.at[b]], add=True)
            if with_deg:
                pltpu.sync_copy(ones, acc2.at[drow.at[b]], add=True)

        for b in range(nbuf):
            fire_g(b, b)

        M = ((NCH - nbuf) // nbuf) * nbuf

        @pl.loop(0, M, step=nbuf)
        def _chunk(j):
            for b in range(nbuf):
                drain(j + b, b)
                fire_g(j + b + nbuf, b)

        for t in range(M, NCH):
            drain(t, t % nbuf)
            if t + nbuf < NCH:
                fire_g(t + nbuf, (t + nbuf) % nbuf)

        plsc.subcore_barrier()
        pltpu.sync_copy(acc.at[pl.ds(base, RPT)], out.at[c, pl.ds(base, RPT)])
        if with_deg:
            pltpu.sync_copy(acc2.at[pl.ds(base, RPT)],
                            outd.at[c, pl.ds(base, RPT)])

    out_type = [jax.ShapeDtypeStruct((NC, NP, d), jnp.float32)]
    if with_deg:
        out_type.append(jax.ShapeDtypeStruct((NC, NP, 16), jnp.float32))
    scratch = [
        pltpu.VMEM_SHARED((NP, d), jnp.float32),
        pltpu.VMEM((NCH, CH), jnp.int32),
        pltpu.VMEM((nbuf, CH), jnp.int32),
        pltpu.VMEM((nbuf, CH), jnp.int32),
    ]
    if with_deg:
        scratch += [
            pltpu.VMEM_SHARED((NP, 16), jnp.float32),
            pltpu.VMEM((CH, 16), jnp.float32),
        ]
    scratch += [pltpu.VMEM((CH, d), jnp.float32) for _ in range(nbuf)]
    scratch += [pltpu.SemaphoreType.DMA for _ in range(nbuf)]

    return pl.kernel(
        body,
        out_type=tuple(out_type) if with_deg else out_type[0],
        mesh=_MESH,
        compiler_params=_SC_PARAMS,
        scratch_types=scratch,
    )


def _deg_body(dst, out, acc, dst_v, ones, sem0, sem1):
    sems = (sem0, sem1)
    c = lax.axis_index("c")
    s = lax.axis_index("s")
    w = c * NS + s
    pltpu.sync_copy(dst.at[w], dst_v)
    _fill(ones, ZCH, 16, 0.0)
    base = s * RPT
    off = 0
    while off < RPT:
        zn = min(ZCH, RPT - off)
        pltpu.sync_copy(ones.at[pl.ds(0, zn)],
                        acc.at[pl.ds(base + off, zn)])
        off += zn
    _fill(ones, CH, 16, 1.0)
    plsc.subcore_barrier()

    def fire(jj, b):
        pltpu.async_copy(ones, acc.at[dst_v.at[jj]], sems[b], add=True)

    def wait(jj, b):
        pltpu.make_async_copy(ones, acc.at[dst_v.at[jj]], sems[b]).wait()

    fire(0, 0)
    fire(1, 1)

    @pl.loop(0, NCH - 2, step=2)
    def _chunk(j):
        for b in range(2):
            wait(j + b, b)
            fire(j + b + 2, b)

    wait(NCH - 2, 0)
    wait(NCH - 1, 1)
    plsc.subcore_barrier()
    pltpu.sync_copy(acc.at[pl.ds(base, RPT)], out.at[c, pl.ds(base, RPT)])


_deg = pl.kernel(
    _deg_body,
    out_type=jax.ShapeDtypeStruct((NC, NP, 16), jnp.float32),
    mesh=_MESH,
    compiler_params=_SC_PARAMS,
    scratch_types=[
        pltpu.VMEM_SHARED((NP, 16), jnp.float32),
        pltpu.VMEM((NCH, CH), jnp.int32),
        pltpu.VMEM((CH, 16), jnp.float32),
        pltpu.SemaphoreType.DMA,
        pltpu.SemaphoreType.DMA,
    ],
)


BR = 2000
GRID = N // BR


def _rows(dd):
    return pl.BlockSpec((BR, dd), lambda i: (i, 0))


def _prows(dd):
    return pl.BlockSpec((2, BR, dd), lambda i: (0, i, 0))


def _full(*shape):
    return pl.BlockSpec(shape, lambda i: (0,) * len(shape))


def _mid1_body(p, pd, w1, b1, w2, y2, invd):
    a = p[...]
    dg = pd[...]
    inv = 1.0 / jnp.maximum(dg[0] + dg[1], 1.0)
    agg = (a[0] + a[1]) * inv[:, :1]
    h1 = jnp.maximum(
        jnp.dot(agg, w1[...], preferred_element_type=jnp.float32)
        + b1[...][None, :], 0.0)
    y2[...] = jnp.dot(h1, w2[...], preferred_element_type=jnp.float32)
    invd[...] = inv


_mid1 = pl.pallas_call(
    _mid1_body, grid=(GRID,),
    in_specs=[_prows(G), _prows(16), _full(G, G), _full(G), _full(G, 64)],
    out_specs=(_rows(64), _rows(16)),
    out_shape=(jax.ShapeDtypeStruct((N, 64), jnp.float32),
               jax.ShapeDtypeStruct((N, 16), jnp.float32)))


def _mid2_body(p, invd, b2, w3, y3):
    a = p[...]
    h2 = jnp.maximum((a[0] + a[1]) * invd[...][:, :1] + b2[...][None, :], 0.0)
    y3[...] = jnp.dot(h2, w3[...], preferred_element_type=jnp.float32)


_mid2 = pl.pallas_call(
    _mid2_body, grid=(GRID,),
    in_specs=[_prows(64), _rows(16), _full(64), _full(64, 64)],
    out_specs=_rows(64),
    out_shape=jax.ShapeDtypeStruct((N, 64), jnp.float32))


def _mid3_body(p, invd, b31, b32, eps, mu, lv, z):
    a = p[...]
    sm = (a[0] + a[1]) * invd[...][:, :1]
    mu_ = sm[:, :32] + b31[...][None, :]
    lv_ = sm[:, 32:] + b32[...][None, :]
    mu[...] = mu_
    lv[...] = lv_
    z[...] = eps[...] * jnp.exp(0.5 * lv_) + mu_


_mid3 = pl.pallas_call(
    _mid3_body, grid=(GRID,),
    in_specs=[_prows(64), _rows(16), _full(32), _full(32), _rows(32)],
    out_specs=(_rows(32), _rows(32), _rows(32)),
    out_shape=(jax.ShapeDtypeStruct((N, 32), jnp.float32),
               jax.ShapeDtypeStruct((N, 32), jnp.float32),
               jax.ShapeDtypeStruct((N, 32), jnp.float32)))


def _make_post(din, dout, act):
    def body(p, invd, w, b, o):
        a = p[...]
        agg = (a[0] + a[1]) * invd[...][:, :1]
        o[...] = act(jnp.dot(agg, w[...], preferred_element_type=jnp.float32)
                     + b[...][None, :])

    return pl.pallas_call(
        body, grid=(GRID,),
        in_specs=[_prows(din), _rows(16), _full(din, dout), _full(dout)],
        out_specs=_rows(dout),
        out_shape=jax.ShapeDtypeStruct((N, dout), jnp.float32))


_relu = lambda t: jnp.maximum(t, 0.0)
_mid4 = _make_post(32, 64, _relu)
_mid5 = _make_post(64, G, _relu)
_fin = _make_post(G, G, jax.nn.sigmoid)


def kernel(x, edge_index, eps, W1, b1, W2, b2, W31, b31, W32, b32,
           W4, b4, W5, b5, W6, b6):
    dst = edge_index[1].reshape(NW, NCH, CH)
    pck = ((edge_index[0] << 14) | edge_index[1]).reshape(NW, NCH, CH)
    W3 = jnp.concatenate([W31, W32], axis=1)

    pdeg = _deg(dst)
    p1 = _make_agg(G)(x, pck)
    y2, invd = _mid1(p1, pdeg, W1, b1, W2)
    p2 = _make_agg(64)(y2, pck)
    y3 = _mid2(p2, invd, b2, W3)
    p3 = _make_agg(64)(y3, pck)
    mu, log_var, z = _mid3(p3, invd, b31, b32, eps)
    p4 = _make_agg(32)(z, pck)
    h4 = _mid4(p4, invd, W4, b4)
    p5 = _make_agg(64)(h4, pck)
    h5 = _mid5(p5, invd, W5, b5)
    p6 = _make_agg(G)(h5, pck)
    recon = _fin(p6, invd, W6, b6)
    return (x, recon, mu, log_var)
